# baseline (speedup 1.0000x reference)
import functools
import jax
import jax.numpy as jnp
from jax import lax
from jax.experimental import pallas as pl
from jax.experimental.pallas import tpu as pltpu
from jax.experimental.pallas import tpu_sc as plsc

_B = 1048576
_NUM_NEURONS = 7
_RAM_SIZE = 256
_NC, _NS, _L = 2, 16, 16
_NW = _NC * _NS
_BPW = _B // _NW
_C = 8192
_NCHUNK = _BPW // _C


def _body(a_hbm, b_hbm, ram_hbm, out_hbm, a_v, b_v, out_v, ram_v, lut_v,
          sa0, sa1, sb0, sb1, so0, so1):
    wid = lax.axis_index("s") * _NC + lax.axis_index("c")
    base = wid * _BPW
    sa = (sa0, sa1)
    sb = (sb0, sb1)
    so = (so0, so1)

    def start_in(g):
        s = g % 2
        ha = pltpu.async_copy(a_hbm.at[pl.ds(base + g * _C, _C)], a_v.at[s], sa[s])
        hb = pltpu.async_copy(b_hbm.at[pl.ds(base + g * _C, _C)], b_v.at[s], sb[s])
        return ha, hb

    inflight = {0: start_in(0)}

    pltpu.sync_copy(ram_hbm, ram_v)
    for j in range(_RAM_SIZE // _L):
        acc = ram_v[0, pl.ds(j * _L, _L)] * 64.0
        for i in range(1, _NUM_NEURONS):
            w = float(1 << (_NUM_NEURONS - 1 - i))
            acc = acc + ram_v[i, pl.ds(j * _L, _L)] * w
        lut_v[pl.ds(j * _L, _L)] = acc

    out_h = {}
    for g in range(_NCHUNK):
        s = g % 2
        ha, hb = inflight.pop(g)
        ha.wait()
        hb.wait()
        if g + 1 < _NCHUNK:
            inflight[g + 1] = start_in(g + 1)
        if g - 2 >= 0:
            out_h.pop(g - 2).wait()

        @plsc.parallel_loop(0, _C, step=_L, unroll=32)
        def _gather(o):
            addr = a_v[s, pl.ds(o, _L)] * 16 + b_v[s, pl.ds(o, _L)]
            out_v[s, pl.ds(o, _L)] = plsc.load_gather(lut_v, [addr])

        out_h[g] = pltpu.async_copy(
            out_v.at[s], out_hbm.at[pl.ds(base + g * _C, _C)], so[s])
    for g in out_h:
        out_h[g].wait()


@jax.jit
def kernel(a_digits, b_digits, mult_ram):
    mesh = plsc.VectorSubcoreMesh(core_axis_name="c", subcore_axis_name="s")
    return pl.kernel(
        _body,
        out_type=jax.ShapeDtypeStruct((_B,), jnp.float32),
        mesh=mesh,
        scratch_types=[
            pltpu.VMEM((2, _C), jnp.int32),
            pltpu.VMEM((2, _C), jnp.int32),
            pltpu.VMEM((2, _C), jnp.float32),
            pltpu.VMEM((_NUM_NEURONS, _RAM_SIZE), jnp.float32),
            pltpu.VMEM((_RAM_SIZE,), jnp.float32),
            pltpu.SemaphoreType.DMA,
            pltpu.SemaphoreType.DMA,
            pltpu.SemaphoreType.DMA,
            pltpu.SemaphoreType.DMA,
            pltpu.SemaphoreType.DMA,
            pltpu.SemaphoreType.DMA,
        ],
        compiler_params=pltpu.CompilerParams(needs_layout_passes=False),
    )(a_digits, b_digits, mult_ram)


# speedup vs baseline: 1.0408x; 1.0408x over previous
import functools
import jax
import jax.numpy as jnp
from jax import lax
from jax.experimental import pallas as pl
from jax.experimental.pallas import tpu as pltpu
from jax.experimental.pallas import tpu_sc as plsc

_B = 1048576
_NUM_NEURONS = 7
_RAM_SIZE = 256
_NC, _NS, _L = 2, 16, 16
_NW = _NC * _NS
_BPW = _B // _NW
_C = 16384
_NCHUNK = _BPW // _C


def _body(a_hbm, b_hbm, ram_hbm, out_hbm, a_v, b_v, out_v, ram_v, lut_v,
          sa0, sa1, sb0, sb1, so0, so1):
    wid = lax.axis_index("s") * _NC + lax.axis_index("c")
    base = wid * _BPW
    sa = (sa0, sa1)
    sb = (sb0, sb1)
    so = (so0, so1)

    def start_in(g):
        s = g % 2
        ha = pltpu.async_copy(a_hbm.at[pl.ds(base + g * _C, _C)], a_v.at[s], sa[s])
        hb = pltpu.async_copy(b_hbm.at[pl.ds(base + g * _C, _C)], b_v.at[s], sb[s])
        return ha, hb

    inflight = {0: start_in(0)}

    pltpu.sync_copy(ram_hbm, ram_v)
    for j in range(_RAM_SIZE // _L):
        acc = ram_v[0, pl.ds(j * _L, _L)] * 64.0
        for i in range(1, _NUM_NEURONS):
            w = float(1 << (_NUM_NEURONS - 1 - i))
            acc = acc + ram_v[i, pl.ds(j * _L, _L)] * w
        lut_v[pl.ds(j * _L, _L)] = acc

    out_h = {}
    for g in range(_NCHUNK):
        s = g % 2
        ha, hb = inflight.pop(g)
        ha.wait()
        hb.wait()
        if g + 1 < _NCHUNK:
            inflight[g + 1] = start_in(g + 1)
        if g - 2 >= 0:
            out_h.pop(g - 2).wait()

        @plsc.parallel_loop(0, _C, step=_L, unroll=16)
        def _gather(o):
            addr = a_v[s, pl.ds(o, _L)] * 16 + b_v[s, pl.ds(o, _L)]
            out_v[s, pl.ds(o, _L)] = plsc.load_gather(lut_v, [addr])

        out_h[g] = pltpu.async_copy(
            out_v.at[s], out_hbm.at[pl.ds(base + g * _C, _C)], so[s])
    for g in out_h:
        out_h[g].wait()


@jax.jit
def kernel(a_digits, b_digits, mult_ram):
    mesh = plsc.VectorSubcoreMesh(core_axis_name="c", subcore_axis_name="s")
    return pl.kernel(
        _body,
        out_type=jax.ShapeDtypeStruct((_B,), jnp.float32),
        mesh=mesh,
        scratch_types=[
            pltpu.VMEM((2, _C), jnp.int32),
            pltpu.VMEM((2, _C), jnp.int32),
            pltpu.VMEM((2, _C), jnp.float32),
            pltpu.VMEM((_NUM_NEURONS, _RAM_SIZE), jnp.float32),
            pltpu.VMEM((_RAM_SIZE,), jnp.float32),
            pltpu.SemaphoreType.DMA,
            pltpu.SemaphoreType.DMA,
            pltpu.SemaphoreType.DMA,
            pltpu.SemaphoreType.DMA,
            pltpu.SemaphoreType.DMA,
            pltpu.SemaphoreType.DMA,
        ],
        compiler_params=pltpu.CompilerParams(needs_layout_passes=False),
    )(a_digits, b_digits, mult_ram)


# speedup vs baseline: 1.0460x; 1.0050x over previous
import functools
import jax
import jax.numpy as jnp
from jax import lax
from jax.experimental import pallas as pl
from jax.experimental.pallas import tpu as pltpu
from jax.experimental.pallas import tpu_sc as plsc

_B = 1048576
_NUM_NEURONS = 7
_RAM_SIZE = 256
_NC, _NS, _L = 2, 16, 16
_NW = _NC * _NS
_BPW = _B // _NW
_C = 16384
_NCHUNK = _BPW // _C


def _body(a_hbm, b_hbm, ram_hbm, out_hbm, a_v, b_v, out_v, ram_v, lut_v,
          sa0, sa1, sb0, sb1, so0, so1):
    wid = lax.axis_index("s") * _NC + lax.axis_index("c")
    base = wid * _BPW
    sa = (sa0, sa1)
    sb = (sb0, sb1)
    so = (so0, so1)

    def start_in(g):
        s = g % 2
        ha = pltpu.async_copy(a_hbm.at[pl.ds(base + g * _C, _C)], a_v.at[s], sa[s])
        hb = pltpu.async_copy(b_hbm.at[pl.ds(base + g * _C, _C)], b_v.at[s], sb[s])
        return ha, hb

    inflight = {0: start_in(0)}

    pltpu.sync_copy(ram_hbm, ram_v)
    for j in range(_RAM_SIZE // _L):
        acc = ram_v[0, pl.ds(j * _L, _L)] * 64.0
        for i in range(1, _NUM_NEURONS):
            w = float(1 << (_NUM_NEURONS - 1 - i))
            acc = acc + ram_v[i, pl.ds(j * _L, _L)] * w
        lut_v[pl.ds(j * _L, _L)] = acc

    out_h = {}
    for g in range(_NCHUNK):
        s = g % 2
        ha, hb = inflight.pop(g)
        ha.wait()
        hb.wait()
        if g + 1 < _NCHUNK:
            inflight[g + 1] = start_in(g + 1)
        if g - 2 >= 0:
            out_h.pop(g - 2).wait()

        @plsc.parallel_loop(0, _C, step=_L, unroll=8)
        def _gather(o):
            addr = a_v[s, pl.ds(o, _L)] * 16 + b_v[s, pl.ds(o, _L)]
            out_v[s, pl.ds(o, _L)] = plsc.load_gather(lut_v, [addr])

        out_h[g] = pltpu.async_copy(
            out_v.at[s], out_hbm.at[pl.ds(base + g * _C, _C)], so[s])
    for g in out_h:
        out_h[g].wait()


@jax.jit
def kernel(a_digits, b_digits, mult_ram):
    mesh = plsc.VectorSubcoreMesh(core_axis_name="c", subcore_axis_name="s")
    return pl.kernel(
        _body,
        out_type=jax.ShapeDtypeStruct((_B,), jnp.float32),
        mesh=mesh,
        scratch_types=[
            pltpu.VMEM((2, _C), jnp.int32),
            pltpu.VMEM((2, _C), jnp.int32),
            pltpu.VMEM((2, _C), jnp.float32),
            pltpu.VMEM((_NUM_NEURONS, _RAM_SIZE), jnp.float32),
            pltpu.VMEM((_RAM_SIZE,), jnp.float32),
            pltpu.SemaphoreType.DMA,
            pltpu.SemaphoreType.DMA,
            pltpu.SemaphoreType.DMA,
            pltpu.SemaphoreType.DMA,
            pltpu.SemaphoreType.DMA,
            pltpu.SemaphoreType.DMA,
        ],
        compiler_params=pltpu.CompilerParams(needs_layout_passes=False),
    )(a_digits, b_digits, mult_ram)


# speedup vs baseline: 1.0475x; 1.0014x over previous
import functools
import jax
import jax.numpy as jnp
from jax import lax
from jax.experimental import pallas as pl
from jax.experimental.pallas import tpu as pltpu
from jax.experimental.pallas import tpu_sc as plsc

_B = 1048576
_NUM_NEURONS = 7
_RAM_SIZE = 256
_NC, _NS, _L = 2, 16, 16
_NW = _NC * _NS
_BPW = _B // _NW
_C = 16384
_NCHUNK = _BPW // _C


def _body(a_hbm, b_hbm, ram_hbm, out_hbm, a_v, b_v, out_v, ram_v, lut_v,
          sa0, sa1, sb0, sb1, so0, so1):
    wid = lax.axis_index("s") * _NC + lax.axis_index("c")
    base = wid * _BPW
    sa = (sa0, sa1)
    sb = (sb0, sb1)
    so = (so0, so1)

    def start_in(g):
        s = g % 2
        ha = pltpu.async_copy(a_hbm.at[pl.ds(base + g * _C, _C)], a_v.at[s], sa[s])
        hb = pltpu.async_copy(b_hbm.at[pl.ds(base + g * _C, _C)], b_v.at[s], sb[s])
        return ha, hb

    inflight = {0: start_in(0)}

    pltpu.sync_copy(ram_hbm, ram_v)
    for j in range(_RAM_SIZE // _L):
        acc = ram_v[0, pl.ds(j * _L, _L)] * 64.0
        for i in range(1, _NUM_NEURONS):
            w = float(1 << (_NUM_NEURONS - 1 - i))
            acc = acc + ram_v[i, pl.ds(j * _L, _L)] * w
        lut_v[pl.ds(j * _L, _L)] = acc

    out_h = {}
    for g in range(_NCHUNK):
        s = g % 2
        ha, hb = inflight.pop(g)
        ha.wait()
        hb.wait()
        if g + 1 < _NCHUNK:
            inflight[g + 1] = start_in(g + 1)
        if g - 2 >= 0:
            out_h.pop(g - 2).wait()

        @plsc.parallel_loop(0, _C, step=_L, unroll=4)
        def _gather(o):
            addr = a_v[s, pl.ds(o, _L)] * 16 + b_v[s, pl.ds(o, _L)]
            out_v[s, pl.ds(o, _L)] = plsc.load_gather(lut_v, [addr])

        out_h[g] = pltpu.async_copy(
            out_v.at[s], out_hbm.at[pl.ds(base + g * _C, _C)], so[s])
    for g in out_h:
        out_h[g].wait()


@jax.jit
def kernel(a_digits, b_digits, mult_ram):
    mesh = plsc.VectorSubcoreMesh(core_axis_name="c", subcore_axis_name="s")
    return pl.kernel(
        _body,
        out_type=jax.ShapeDtypeStruct((_B,), jnp.float32),
        mesh=mesh,
        scratch_types=[
            pltpu.VMEM((2, _C), jnp.int32),
            pltpu.VMEM((2, _C), jnp.int32),
            pltpu.VMEM((2, _C), jnp.float32),
            pltpu.VMEM((_NUM_NEURONS, _RAM_SIZE), jnp.float32),
            pltpu.VMEM((_RAM_SIZE,), jnp.float32),
            pltpu.SemaphoreType.DMA,
            pltpu.SemaphoreType.DMA,
            pltpu.SemaphoreType.DMA,
            pltpu.SemaphoreType.DMA,
            pltpu.SemaphoreType.DMA,
            pltpu.SemaphoreType.DMA,
        ],
        compiler_params=pltpu.CompilerParams(needs_layout_passes=False),
    )(a_digits, b_digits, mult_ram)
